# Initial kernel scaffold; baseline (speedup 1.0000x reference)
import jax
import jax.numpy as jnp
from jax.experimental import pallas as pl


def kernel(x_user, x_merchant, edge_index, edge_index_rev, edge_attr, params):
    raise NotImplementedError("write your pallas kernel here")


# speedup vs baseline: 1.4965x; 1.4965x over previous
import functools

import jax
import jax.numpy as jnp
from jax import lax
from jax.experimental import pallas as pl
from jax.experimental.pallas import tpu as pltpu
from jax.experimental.pallas import tpu_sc as plsc

H = 128


def _mm(a, b):
    return jnp.dot(a, b, preferred_element_type=jnp.float32)


def _node_body(xu, xm,
               ueW1, ueb1, ueW2, ueb2,
               meW1, meb1, meW2, meb2,
               impW1, impb1, impW2, impb2,
               intW1, intb1, intW2, intb2,
               npW, npb, W1a, W1b,
               eeW2, eeb2, W1c, mlpb1,
               table_u_ref, table_m_ref, wec_ref, btot_ref):
    relu = jax.nn.relu

    def pipe(x, Wa, ba, Wb, bb, Wproj):
        h = _mm(relu(_mm(x, Wa) + ba), Wb) + bb
        imp = jax.nn.sigmoid(
            _mm(relu(_mm(h, impW1[...]) + impb1[...]), impW2[...]) + impb2[...])
        t = _mm(relu(_mm(h, intW1[...]) + intb1[...]), intW2[...]) + intb2[...]
        h = h + t * imp
        h = _mm(h, npW[...]) + npb[...]
        return _mm(h, Wproj)

    table_u_ref[...] = pipe(xu[...], ueW1[...], ueb1[...], ueW2[...], ueb2[...], W1a[...])
    table_m_ref[...] = pipe(xm[...], meW1[...], meb1[...], meW2[...], meb2[...], W1b[...])
    wec_ref[...] = _mm(eeW2[...], W1c[...])
    btot_ref[...] = mlpb1[...] + _mm(eeb2[...], W1c[...])


def _edge_body(srcr, dstr, ea, eeW1, eeb1, wec, btot, W2, b2, W3, b3, out_ref):
    relu = jax.nn.relu
    e1 = relu(_mm(ea[...], eeW1[...]) + eeb1[...])
    z = relu(srcr[...] + dstr[...] + _mm(e1, wec[...]) + btot[...])
    h2 = relu(_mm(z, W2[...]) + b2[...])
    out_ref[...] = _mm(h2, W3[...]) + b3[...]


def _make_gather(n_nodes, b_pad, per_w, ch, nc, ns):
    n_ch = per_w // ch
    mesh = plsc.VectorSubcoreMesh(core_axis_name="c", subcore_axis_name="s")

    @functools.partial(
        pl.kernel,
        out_type=[jax.ShapeDtypeStruct((b_pad, H), jnp.float32),
                  jax.ShapeDtypeStruct((b_pad, H), jnp.float32)],
        mesh=mesh,
        scratch_types=[
            pltpu.VMEM((ch,), jnp.int32),
            pltpu.VMEM((ch,), jnp.int32),
            pltpu.VMEM((ch, H), jnp.float32),
            pltpu.VMEM((ch, H), jnp.float32),
            pltpu.SemaphoreType.DMA,
            pltpu.SemaphoreType.DMA,
        ],
    )
    def gather_k(table_hbm, src_hbm, dst_hbm, outu_hbm, outm_hbm,
                 idx_u, idx_m, rows_u, rows_m, sem_u, sem_m):
        wid = lax.axis_index("s") * nc + lax.axis_index("c")
        base = pl.multiple_of(wid * per_w, ch)

        def body(i, _):
            off = pl.multiple_of(base + i * ch, ch)
            pltpu.sync_copy(src_hbm.at[pl.ds(off, ch)], idx_u)
            pltpu.sync_copy(dst_hbm.at[pl.ds(off, ch)], idx_m)
            cu = pltpu.async_copy(table_hbm.at[idx_u], rows_u, sem_u)
            cm = pltpu.async_copy(table_hbm.at[idx_m], rows_m, sem_m)
            cu.wait()
            cm.wait()
            pltpu.sync_copy(rows_u, outu_hbm.at[pl.ds(off, ch)])
            pltpu.sync_copy(rows_m, outm_hbm.at[pl.ds(off, ch)])
            return 0

        lax.fori_loop(0, n_ch, body, 0)

    return gather_k


def kernel(x_user, x_merchant, edge_index, edge_index_rev, edge_attr, params):
    del edge_index_rev
    p = params
    n_u = x_user.shape[0]
    n_m = x_merchant.shape[0]
    n_edges = edge_index.shape[1]
    e_dim = edge_attr.shape[1]

    def row(v):
        return v.reshape(1, -1)

    W1a = p['mlp_W1'][:H]
    W1b = p['mlp_W1'][H:2 * H]
    W1c = p['mlp_W1'][2 * H:]

    nb = 5
    blk_u = n_u // nb
    blk_m = n_m // nb

    def full(shape):
        return pl.BlockSpec(shape, lambda i: tuple(0 for _ in shape))

    w_specs = [
        full((H, H)), full((1, H)), full((H, H)), full((1, H)),
        full((H, H)), full((1, H)), full((H, H)), full((1, H)),
        full((H, H // 2)), full((1, H // 2)), full((H // 2, 1)), full((1, 1)),
        full((H, H)), full((1, H)), full((H, H)), full((1, H)),
        full((H, H)), full((1, H)), full((H, H)), full((H, H)),
        full((H, H)), full((1, H)), full((H, H)), full((1, H)),
    ]
    node_out = pl.pallas_call(
        _node_body,
        grid=(nb,),
        in_specs=[pl.BlockSpec((blk_u, H), lambda i: (i, 0)),
                  pl.BlockSpec((blk_m, H), lambda i: (i, 0))] + w_specs,
        out_specs=[pl.BlockSpec((blk_u, H), lambda i: (i, 0)),
                   pl.BlockSpec((blk_m, H), lambda i: (i, 0)),
                   full((H, H)), full((1, H))],
        out_shape=[jax.ShapeDtypeStruct((n_u, H), jnp.float32),
                   jax.ShapeDtypeStruct((n_m, H), jnp.float32),
                   jax.ShapeDtypeStruct((H, H), jnp.float32),
                   jax.ShapeDtypeStruct((1, H), jnp.float32)],
    )(x_user, x_merchant,
      p['ue_W1'], row(p['ue_b1']), p['ue_W2'], row(p['ue_b2']),
      p['me_W1'], row(p['me_b1']), p['me_W2'], row(p['me_b2']),
      p['imp_W1'], row(p['imp_b1']), p['imp_W2'], row(p['imp_b2']),
      p['int_W1'], row(p['int_b1']), p['int_W2'], row(p['int_b2']),
      p['np_W'], row(p['np_b']), W1a, W1b,
      p['ee_W2'], row(p['ee_b2']), W1c, row(p['mlp_b1']))
    table_u, table_m, wec, btot = node_out
    table = jnp.concatenate([table_u, table_m], axis=0)

    info = plsc.get_sparse_core_info()
    nc, ns = info.num_cores, info.num_subcores
    nw = nc * ns
    ch = 128
    per_w = -(-n_edges // (nw * ch)) * ch
    b_pad = per_w * nw

    src = jnp.pad(edge_index[0].astype(jnp.int32), (0, b_pad - n_edges))
    dst = jnp.pad(edge_index[1].astype(jnp.int32) + n_u, (0, b_pad - n_edges),
                  constant_values=n_u)

    gather_k = _make_gather(n_u + n_m, b_pad, per_w, ch, nc, ns)
    src_rows, dst_rows = gather_k(table, src, dst)

    eb = 4096
    n_eb = b_pad // eb
    ea_pad = jnp.pad(edge_attr, ((0, b_pad - n_edges), (0, 0)))

    logits_pad = pl.pallas_call(
        _edge_body,
        grid=(n_eb,),
        in_specs=[pl.BlockSpec((eb, H), lambda i: (i, 0)),
                  pl.BlockSpec((eb, H), lambda i: (i, 0)),
                  pl.BlockSpec((eb, e_dim), lambda i: (i, 0)),
                  full((e_dim, H)), full((1, H)),
                  full((H, H)), full((1, H)),
                  full((H, H // 2)), full((1, H // 2)),
                  full((H // 2, 2)), full((1, 2))],
        out_specs=pl.BlockSpec((eb, 2), lambda i: (i, 0)),
        out_shape=jax.ShapeDtypeStruct((b_pad, 2), jnp.float32),
    )(src_rows, dst_rows, ea_pad,
      p['ee_W1'], row(p['ee_b1']), wec, btot,
      p['mlp_W2'], row(p['mlp_b2']), p['mlp_W3'], row(p['mlp_b3']))

    return logits_pad[:n_edges]
